# baseline (speedup 1.0000x reference)
import jax
import jax.numpy as jnp
from jax import lax
from jax.experimental import pallas as pl
from jax.experimental.pallas import tpu as pltpu
from jax.experimental.pallas import tpu_sc as plsc

N_POINTS = 4096
N_BATCH = 4
QSPLIT = 3712
SC_Q = N_POINTS - QSPLIT

L = 16
QV = 2
NUM_CORES = 2
NUM_SUBCORES = 16
NUM_TILES = NUM_CORES * NUM_SUBCORES


def _sc_sweep(q_hbm, k_hbm, out_hbm, kpx, kpy, kpz, kn,
              qsx, qsy, qsz, o_v, qbase, kbase, obase, qlen, klen):
    for d in range(3):
        pltpu.sync_copy(k_hbm.at[pl.ds(kbase + d * N_POINTS, klen)],
                        [kpx, kpy, kpz][d].at[pl.ds(0, klen)])
        pltpu.sync_copy(q_hbm.at[pl.ds(qbase + d * N_POINTS, qlen)],
                        [qsx, qsy, qsz][d].at[pl.ds(0, qlen)])

    def pre_body(t, carry):
        sl = pl.ds(t * L, L)
        kx = kpx[sl]
        ky = kpy[sl]
        kz = kpz[sl]
        kn[sl] = kx * kx + ky * ky + kz * kz
        kpx[sl] = kx * (-2.0)
        kpy[sl] = ky * (-2.0)
        kpz[sl] = kz * (-2.0)
        return carry

    lax.fori_loop(0, klen // L, pre_body, 0)

    def group_body(g, carry):
        base = g * (QV * L)
        qx = [qsx[pl.ds(base + i * L, L)] for i in range(QV)]
        qy = [qsy[pl.ds(base + i * L, L)] for i in range(QV)]
        qz = [qsz[pl.ds(base + i * L, L)] for i in range(QV)]
        qn = [qx[i] * qx[i] + qy[i] * qy[i] + qz[i] * qz[i]
              for i in range(QV)]
        inf = jnp.full((L,), jnp.inf, jnp.float32)

        def key_body(t, accs):
            sl = pl.ds(t * L, L)
            kxv = kpx[sl]
            kyv = kpy[sl]
            kzv = kpz[sl]
            knv = kn[sl]
            accs = list(accs)
            for u in range(L):
                idx = jnp.full((L,), u, jnp.int32)
                bkx = kxv.at[idx].get(mode="promise_in_bounds")
                bky = kyv.at[idx].get(mode="promise_in_bounds")
                bkz = kzv.at[idx].get(mode="promise_in_bounds")
                bkn = knv.at[idx].get(mode="promise_in_bounds")
                for i in range(QV):
                    d = (qx[i] * bkx + qy[i] * bky) + (qz[i] * bkz + bkn)
                    accs[i] = jnp.minimum(accs[i], d)
            return tuple(accs)

        accs = lax.fori_loop(0, klen // L, key_body, (inf,) * QV)
        for i in range(QV):
            o_v[pl.ds(base + i * L, L)] = accs[i] + qn[i]
        return carry

    lax.fori_loop(0, qlen // (QV * L), group_body, 0)

    pltpu.sync_copy(o_v.at[pl.ds(0, qlen)], out_hbm.at[pl.ds(obase, qlen)])


def _sc_body(x1_hbm, x2_hbm, out_hbm,
             kpx, kpy, kpz, kn, qsx, qsy, qsz, o_v):
    wid = lax.axis_index("s") * NUM_CORES + lax.axis_index("c")
    item = wid // 4
    s = wid % 4
    batch = item % N_BATCH
    q1 = SC_Q // 4
    q2 = N_POINTS // 4

    @pl.when(item < N_BATCH)
    def _dir1():
        _sc_sweep(x1_hbm, x2_hbm, out_hbm, kpx, kpy, kpz, kn,
                  qsx, qsy, qsz, o_v,
                  qbase=batch * 3 * N_POINTS + QSPLIT + s * q1,
                  kbase=batch * 3 * N_POINTS,
                  obase=batch * SC_Q + s * q1,
                  qlen=q1, klen=N_POINTS)

    @pl.when(item >= N_BATCH)
    def _dir2():
        _sc_sweep(x2_hbm, x1_hbm, out_hbm, kpx, kpy, kpz, kn,
                  qsx, qsy, qsz, o_v,
                  qbase=batch * 3 * N_POINTS + s * q2,
                  kbase=batch * 3 * N_POINTS + QSPLIT,
                  obase=N_BATCH * SC_Q + batch * N_POINTS + s * q2,
                  qlen=q2, klen=SC_Q)


@jax.jit
def _chamfer_sc(x1t, x2t):
    mesh = plsc.VectorSubcoreMesh(core_axis_name="c", subcore_axis_name="s",
                                  num_cores=NUM_CORES,
                                  num_subcores=NUM_SUBCORES)
    return pl.kernel(
        _sc_body,
        out_type=jax.ShapeDtypeStruct(
            (N_BATCH * SC_Q + N_BATCH * N_POINTS,), jnp.float32),
        mesh=mesh,
        scratch_types=[
            pltpu.VMEM((N_POINTS,), jnp.float32),
            pltpu.VMEM((N_POINTS,), jnp.float32),
            pltpu.VMEM((N_POINTS,), jnp.float32),
            pltpu.VMEM((N_POINTS,), jnp.float32),
            pltpu.VMEM((N_POINTS // 4,), jnp.float32),
            pltpu.VMEM((N_POINTS // 4,), jnp.float32),
            pltpu.VMEM((N_POINTS // 4,), jnp.float32),
            pltpu.VMEM((N_POINTS // 4,), jnp.float32),
        ],
    )(x1t.reshape(-1), x2t.reshape(-1))


RBLK = 928


def _tc_body(qa_ref, ks_ref, o1_ref, o2_ref):
    r = pl.program_id(1)
    qa = qa_ref[0]
    ks = ks_ref[0]
    d = None
    for c in range(3):
        diff = qa[:, c:c + 1] - ks[c:c + 1, :]
        sq = diff * diff
        d = sq if d is None else d + sq
    o1_ref[0] = jnp.min(d, axis=1, keepdims=True)
    row = jnp.min(d, axis=0, keepdims=True)

    @pl.when(r == 0)
    def _init():
        o2_ref[0] = row

    @pl.when(r > 0)
    def _acc():
        o2_ref[0] = jnp.minimum(o2_ref[0], row)


@jax.jit
def _chamfer_tc(qa, ks):
    grid = (N_BATCH, QSPLIT // RBLK)
    out1, out2 = pl.pallas_call(
        _tc_body,
        grid=grid,
        in_specs=[
            pl.BlockSpec((1, RBLK, 8), lambda c, r: (c, r, 0)),
            pl.BlockSpec((1, 8, N_POINTS), lambda c, r: (c, 0, 0)),
        ],
        out_specs=[
            pl.BlockSpec((1, RBLK, 1), lambda c, r: (c, r, 0)),
            pl.BlockSpec((1, 1, N_POINTS), lambda c, r: (c, 0, 0)),
        ],
        out_shape=[
            jax.ShapeDtypeStruct((N_BATCH, QSPLIT, 1), jnp.float32),
            jax.ShapeDtypeStruct((N_BATCH, 1, N_POINTS), jnp.float32),
        ],
    )(qa, ks)
    return out1.reshape(N_BATCH, QSPLIT), out2.reshape(N_BATCH, N_POINTS)


def kernel(input1, input2):
    x1t = jnp.transpose(input1, (0, 2, 1))
    x2t = jnp.transpose(input2, (0, 2, 1))
    sc_out = _chamfer_sc(x1t, x2t)
    x1a = jnp.pad(input1[:, :QSPLIT, :], ((0, 0), (0, 0), (0, 5)))
    x2p = jnp.pad(x2t, ((0, 0), (0, 5), (0, 0)))
    d1_head, d2_tc = _chamfer_tc(x1a, x2p)
    d1_tail = sc_out[:N_BATCH * SC_Q].reshape(N_BATCH, SC_Q)
    d2_sc = sc_out[N_BATCH * SC_Q:].reshape(N_BATCH, N_POINTS)
    dist1 = jnp.concatenate([d1_head, d1_tail], axis=1)
    dist2 = jnp.minimum(d2_tc, d2_sc)
    return dist1, dist2


# speedup vs baseline: 1.0608x; 1.0608x over previous
import jax
import jax.numpy as jnp
from jax import lax
from jax.experimental import pallas as pl
from jax.experimental.pallas import tpu as pltpu
from jax.experimental.pallas import tpu_sc as plsc

N_POINTS = 4096
N_BATCH = 4
QSPLIT = 3840
SC_Q = N_POINTS - QSPLIT

L = 16
QV = 2
NUM_CORES = 2
NUM_SUBCORES = 16
NUM_TILES = NUM_CORES * NUM_SUBCORES


def _sc_sweep(q_hbm, k_hbm, out_hbm, kpx, kpy, kpz, kn,
              qsx, qsy, qsz, o_v, qbase, kbase, obase, qlen, klen):
    for d in range(3):
        pltpu.sync_copy(k_hbm.at[pl.ds(kbase + d * N_POINTS, klen)],
                        [kpx, kpy, kpz][d].at[pl.ds(0, klen)])
        pltpu.sync_copy(q_hbm.at[pl.ds(qbase + d * N_POINTS, qlen)],
                        [qsx, qsy, qsz][d].at[pl.ds(0, qlen)])

    def pre_body(t, carry):
        sl = pl.ds(t * L, L)
        kx = kpx[sl]
        ky = kpy[sl]
        kz = kpz[sl]
        kn[sl] = kx * kx + ky * ky + kz * kz
        kpx[sl] = kx * (-2.0)
        kpy[sl] = ky * (-2.0)
        kpz[sl] = kz * (-2.0)
        return carry

    lax.fori_loop(0, klen // L, pre_body, 0)

    def group_body(g, carry):
        base = g * (QV * L)
        qx = [qsx[pl.ds(base + i * L, L)] for i in range(QV)]
        qy = [qsy[pl.ds(base + i * L, L)] for i in range(QV)]
        qz = [qsz[pl.ds(base + i * L, L)] for i in range(QV)]
        qn = [qx[i] * qx[i] + qy[i] * qy[i] + qz[i] * qz[i]
              for i in range(QV)]
        inf = jnp.full((L,), jnp.inf, jnp.float32)

        def key_body(t, accs):
            sl = pl.ds(t * L, L)
            kxv = kpx[sl]
            kyv = kpy[sl]
            kzv = kpz[sl]
            knv = kn[sl]
            accs = list(accs)
            for u in range(L):
                idx = jnp.full((L,), u, jnp.int32)
                bkx = kxv.at[idx].get(mode="promise_in_bounds")
                bky = kyv.at[idx].get(mode="promise_in_bounds")
                bkz = kzv.at[idx].get(mode="promise_in_bounds")
                bkn = knv.at[idx].get(mode="promise_in_bounds")
                for i in range(QV):
                    d = (qx[i] * bkx + qy[i] * bky) + (qz[i] * bkz + bkn)
                    accs[i] = jnp.minimum(accs[i], d)
            return tuple(accs)

        accs = lax.fori_loop(0, klen // L, key_body, (inf,) * QV)
        for i in range(QV):
            o_v[pl.ds(base + i * L, L)] = accs[i] + qn[i]
        return carry

    lax.fori_loop(0, qlen // (QV * L), group_body, 0)

    pltpu.sync_copy(o_v.at[pl.ds(0, qlen)], out_hbm.at[pl.ds(obase, qlen)])


def _sc_body(x1_hbm, x2_hbm, out_hbm,
             kpx, kpy, kpz, kn, qsx, qsy, qsz, o_v):
    wid = lax.axis_index("s") * NUM_CORES + lax.axis_index("c")
    item = wid // 4
    s = wid % 4
    batch = item % N_BATCH
    q1 = SC_Q // 4
    q2 = N_POINTS // 4

    @pl.when(item < N_BATCH)
    def _dir1():
        _sc_sweep(x1_hbm, x2_hbm, out_hbm, kpx, kpy, kpz, kn,
                  qsx, qsy, qsz, o_v,
                  qbase=batch * 3 * N_POINTS + QSPLIT + s * q1,
                  kbase=batch * 3 * N_POINTS,
                  obase=batch * SC_Q + s * q1,
                  qlen=q1, klen=N_POINTS)

    @pl.when(item >= N_BATCH)
    def _dir2():
        _sc_sweep(x2_hbm, x1_hbm, out_hbm, kpx, kpy, kpz, kn,
                  qsx, qsy, qsz, o_v,
                  qbase=batch * 3 * N_POINTS + s * q2,
                  kbase=batch * 3 * N_POINTS + QSPLIT,
                  obase=N_BATCH * SC_Q + batch * N_POINTS + s * q2,
                  qlen=q2, klen=SC_Q)


@jax.jit
def _chamfer_sc(x1t, x2t):
    mesh = plsc.VectorSubcoreMesh(core_axis_name="c", subcore_axis_name="s",
                                  num_cores=NUM_CORES,
                                  num_subcores=NUM_SUBCORES)
    return pl.kernel(
        _sc_body,
        out_type=jax.ShapeDtypeStruct(
            (N_BATCH * SC_Q + N_BATCH * N_POINTS,), jnp.float32),
        mesh=mesh,
        scratch_types=[
            pltpu.VMEM((N_POINTS,), jnp.float32),
            pltpu.VMEM((N_POINTS,), jnp.float32),
            pltpu.VMEM((N_POINTS,), jnp.float32),
            pltpu.VMEM((N_POINTS,), jnp.float32),
            pltpu.VMEM((N_POINTS // 4,), jnp.float32),
            pltpu.VMEM((N_POINTS // 4,), jnp.float32),
            pltpu.VMEM((N_POINTS // 4,), jnp.float32),
            pltpu.VMEM((N_POINTS // 4,), jnp.float32),
        ],
    )(x1t.reshape(-1), x2t.reshape(-1))


RBLK = 1280


def _tc_body(qa_ref, ks_ref, o1_ref, o2_ref):
    r = pl.program_id(1)
    qa = qa_ref[0]
    ks = ks_ref[0]
    d = None
    for c in range(3):
        diff = qa[:, c:c + 1] - ks[c:c + 1, :]
        sq = diff * diff
        d = sq if d is None else d + sq
    o1_ref[0] = jnp.min(d, axis=1, keepdims=True)
    row = jnp.min(d, axis=0, keepdims=True)

    @pl.when(r == 0)
    def _init():
        o2_ref[0] = row

    @pl.when(r > 0)
    def _acc():
        o2_ref[0] = jnp.minimum(o2_ref[0], row)


@jax.jit
def _chamfer_tc(qa, ks):
    grid = (N_BATCH, QSPLIT // RBLK)
    out1, out2 = pl.pallas_call(
        _tc_body,
        grid=grid,
        in_specs=[
            pl.BlockSpec((1, RBLK, 8), lambda c, r: (c, r, 0)),
            pl.BlockSpec((1, 8, N_POINTS), lambda c, r: (c, 0, 0)),
        ],
        out_specs=[
            pl.BlockSpec((1, RBLK, 1), lambda c, r: (c, r, 0)),
            pl.BlockSpec((1, 1, N_POINTS), lambda c, r: (c, 0, 0)),
        ],
        out_shape=[
            jax.ShapeDtypeStruct((N_BATCH, QSPLIT, 1), jnp.float32),
            jax.ShapeDtypeStruct((N_BATCH, 1, N_POINTS), jnp.float32),
        ],
    )(qa, ks)
    return out1.reshape(N_BATCH, QSPLIT), out2.reshape(N_BATCH, N_POINTS)


def kernel(input1, input2):
    x1t = jnp.transpose(input1, (0, 2, 1))
    x2t = jnp.transpose(input2, (0, 2, 1))
    sc_out = _chamfer_sc(x1t, x2t)
    x1a = jnp.pad(input1[:, :QSPLIT, :], ((0, 0), (0, 0), (0, 5)))
    x2p = jnp.pad(x2t, ((0, 0), (0, 5), (0, 0)))
    d1_head, d2_tc = _chamfer_tc(x1a, x2p)
    d1_tail = sc_out[:N_BATCH * SC_Q].reshape(N_BATCH, SC_Q)
    d2_sc = sc_out[N_BATCH * SC_Q:].reshape(N_BATCH, N_POINTS)
    dist1 = jnp.concatenate([d1_head, d1_tail], axis=1)
    dist2 = jnp.minimum(d2_tc, d2_sc)
    return dist1, dist2
